# baseline (speedup 1.0000x reference)
import jax
import jax.numpy as jnp
from jax.experimental import pallas as pl
from jax.experimental.pallas import tpu as pltpu

_G = 128


def _gcn_block(x_ref, adj_ref, w_ref, b_ref, out_ref):
    g, n, d = x_ref.shape
    o = w_ref.shape[1]
    x = x_ref[...].reshape(g * n, d)
    xw = jnp.dot(x, w_ref[...], preferred_element_type=jnp.float32)
    adj_f = adj_ref[...].astype(jnp.float32).reshape(g, n, n)
    deg = jax.lax.dot_general(
        adj_f, jnp.ones((g, n, o), jnp.float32), (((2,), (1,)), ((0,), (0,))),
        preferred_element_type=jnp.float32)
    dinv = jax.lax.rsqrt(deg + 1.0)
    xwn = xw.reshape(g, n, o) * dinv
    agg = jax.lax.dot_general(
        adj_f, xwn, (((2,), (1,)), ((0,), (0,))),
        preferred_element_type=jnp.float32) + xwn
    out = agg * dinv + b_ref[0][None, None, :]
    out_ref[...] = out.reshape(g, n * o)


def kernel(node_states, adj, W_gnn, b_gnn):
    b, n, d = node_states.shape
    o = W_gnn.shape[1]
    out = pl.pallas_call(
        _gcn_block,
        grid=(b // _G,),
        in_specs=[
            pl.BlockSpec((_G, n, d), lambda i: (i, 0, 0)),
            pl.BlockSpec((_G, n * n), lambda i: (i, 0)),
            pl.BlockSpec((d, o), lambda i: (0, 0)),
            pl.BlockSpec((1, o), lambda i: (0, 0)),
        ],
        out_specs=pl.BlockSpec((_G, n * o), lambda i: (i, 0)),
        out_shape=jax.ShapeDtypeStruct((b, n * o), jnp.float32),
        compiler_params=pltpu.CompilerParams(
            dimension_semantics=("parallel",)),
    )(node_states, adj.reshape(b, n * n), W_gnn, b_gnn.reshape(1, o))
    return out


# speedup vs baseline: 1.0687x; 1.0687x over previous
import jax
import jax.numpy as jnp
from jax.experimental import pallas as pl
from jax.experimental.pallas import tpu as pltpu

_G = 256


def _gcn_block(x_ref, adj_ref, w_ref, b_ref, out_ref):
    g, n, d = x_ref.shape
    o = w_ref.shape[1]
    x = x_ref[...].reshape(g * n, d)
    xw = jnp.dot(x, w_ref[...], preferred_element_type=jnp.float32)
    adj_h = adj_ref[...].astype(jnp.bfloat16).reshape(g, n, n)
    deg = jax.lax.dot_general(
        adj_h, jnp.ones((g, n, o), jnp.bfloat16), (((2,), (1,)), ((0,), (0,))),
        preferred_element_type=jnp.float32)
    dinv = jax.lax.rsqrt(deg + 1.0)
    xwn = xw.reshape(g, n, o) * dinv
    agg = jax.lax.dot_general(
        adj_h, xwn.astype(jnp.bfloat16), (((2,), (1,)), ((0,), (0,))),
        preferred_element_type=jnp.float32) + xwn
    out = agg * dinv + b_ref[0][None, None, :]
    out_ref[...] = out.reshape(g, n * o)


def kernel(node_states, adj, W_gnn, b_gnn):
    b, n, d = node_states.shape
    o = W_gnn.shape[1]
    out = pl.pallas_call(
        _gcn_block,
        grid=(b // _G,),
        in_specs=[
            pl.BlockSpec((_G, n, d), lambda i: (i, 0, 0)),
            pl.BlockSpec((_G, n * n), lambda i: (i, 0)),
            pl.BlockSpec((d, o), lambda i: (0, 0)),
            pl.BlockSpec((1, o), lambda i: (0, 0)),
        ],
        out_specs=pl.BlockSpec((_G, n * o), lambda i: (i, 0)),
        out_shape=jax.ShapeDtypeStruct((b, n * o), jnp.float32),
        compiler_params=pltpu.CompilerParams(
            dimension_semantics=("parallel",)),
    )(node_states, adj.reshape(b, n * n), W_gnn, b_gnn.reshape(1, o))
    return out


# speedup vs baseline: 1.1796x; 1.1038x over previous
import jax
import jax.numpy as jnp
from jax.experimental import pallas as pl
from jax.experimental.pallas import tpu as pltpu

_G = 256


def _gcn_block(x_ref, adj_ref, w_ref, b_ref, out_ref):
    g, n, d = x_ref.shape
    o = w_ref.shape[1]
    x = x_ref[...].reshape(g * n, d)
    xw = jnp.dot(x, w_ref[...], preferred_element_type=jnp.float32)
    adj_h = adj_ref[...].astype(jnp.bfloat16).reshape(g, n, n)
    deg = jax.lax.dot_general(
        adj_h, jnp.ones((g, n, o), jnp.bfloat16), (((2,), (1,)), ((0,), (0,))),
        preferred_element_type=jnp.float32)
    dinv = jax.lax.rsqrt(deg + 1.0)
    xwn = xw.reshape(g, n, o) * dinv
    agg = jax.lax.dot_general(
        adj_h, xwn.astype(jnp.bfloat16), (((2,), (1,)), ((0,), (0,))),
        preferred_element_type=jnp.float32) + xwn
    out = agg * dinv + b_ref[0][None, None, :]
    out_ref[...] = out.astype(jnp.bfloat16).reshape(g, n * o).astype(jnp.float32)


def kernel(node_states, adj, W_gnn, b_gnn):
    b, n, d = node_states.shape
    o = W_gnn.shape[1]
    out = pl.pallas_call(
        _gcn_block,
        grid=(b // _G,),
        in_specs=[
            pl.BlockSpec((_G, n, d), lambda i: (i, 0, 0)),
            pl.BlockSpec((_G, n * n), lambda i: (i, 0)),
            pl.BlockSpec((d, o), lambda i: (0, 0)),
            pl.BlockSpec((1, o), lambda i: (0, 0)),
        ],
        out_specs=pl.BlockSpec((_G, n * o), lambda i: (i, 0)),
        out_shape=jax.ShapeDtypeStruct((b, n * o), jnp.float32),
        compiler_params=pltpu.CompilerParams(
            dimension_semantics=("parallel",)),
    )(node_states, adj.reshape(b, n * n), W_gnn, b_gnn.reshape(1, o))
    return out


# speedup vs baseline: 1.2489x; 1.0587x over previous
import jax
import jax.numpy as jnp
from jax.experimental import pallas as pl
from jax.experimental.pallas import tpu as pltpu

_G = 256


def _gcn_block(x_ref, adj_ref, w_ref, diag_ref, k_ref, mj_ref, mo_ref,
               bflat_ref, out_ref):
    g, n, d = x_ref.shape
    o = w_ref.shape[1]
    x = x_ref[...].reshape(g * n, d)
    xw = jnp.dot(x, w_ref[...], preferred_element_type=jnp.float32)
    ah = adj_ref[...].astype(jnp.bfloat16) + diag_ref[0][None, :]
    deg = jnp.dot(ah, k_ref[...], preferred_element_type=jnp.float32)
    s = jax.lax.rsqrt(deg).astype(jnp.bfloat16)
    vj = jnp.dot(s, mj_ref[...], preferred_element_type=jnp.float32)
    ahn = ah * vj.astype(jnp.bfloat16)
    norm3 = ahn.reshape(g, n, n)
    xw3 = xw.reshape(g, n, o).astype(jnp.bfloat16)
    agg = jax.lax.dot_general(
        norm3, xw3, (((2,), (1,)), ((0,), (0,))),
        preferred_element_type=jnp.float32)
    og = agg.astype(jnp.bfloat16).reshape(g, n * o)
    vi = jnp.dot(s, mo_ref[...], preferred_element_type=jnp.float32)
    out_ref[...] = og.astype(jnp.float32) * vi + bflat_ref[0][None, :]


def kernel(node_states, adj, W_gnn, b_gnn):
    b, n, d = node_states.shape
    o = W_gnn.shape[1]
    nn, no = n * n, n * o
    cc = jnp.arange(nn, dtype=jnp.int32)
    kk = jnp.arange(n, dtype=jnp.int32)
    co = jnp.arange(no, dtype=jnp.int32)
    diag = (cc // n == cc % n).astype(jnp.bfloat16).reshape(1, nn)
    k_mat = (cc[:, None] // n == kk[None, :]).astype(jnp.bfloat16)
    mj = (kk[:, None] == cc[None, :] % n).astype(jnp.bfloat16)
    mo = (kk[:, None] == co[None, :] // o).astype(jnp.bfloat16)
    b_flat = jnp.tile(b_gnn, n).reshape(1, no)
    out = pl.pallas_call(
        _gcn_block,
        grid=(b // _G,),
        in_specs=[
            pl.BlockSpec((_G, n, d), lambda i: (i, 0, 0)),
            pl.BlockSpec((_G, nn), lambda i: (i, 0)),
            pl.BlockSpec((d, o), lambda i: (0, 0)),
            pl.BlockSpec((1, nn), lambda i: (0, 0)),
            pl.BlockSpec((nn, n), lambda i: (0, 0)),
            pl.BlockSpec((n, nn), lambda i: (0, 0)),
            pl.BlockSpec((n, no), lambda i: (0, 0)),
            pl.BlockSpec((1, no), lambda i: (0, 0)),
        ],
        out_specs=pl.BlockSpec((_G, no), lambda i: (i, 0)),
        out_shape=jax.ShapeDtypeStruct((b, no), jnp.float32),
        compiler_params=pltpu.CompilerParams(
            dimension_semantics=("parallel",)),
    )(node_states, adj.reshape(b, nn), W_gnn, diag, k_mat, mj, mo, b_flat)
    return out


# speedup vs baseline: 1.3233x; 1.0596x over previous
import jax
import jax.numpy as jnp
from jax.experimental import pallas as pl
from jax.experimental.pallas import tpu as pltpu

_G = 512


def _gcn_block(x_ref, adj_ref, w_ref, diag_ref, k_ref, mj_ref, mo_ref,
               bflat_ref, out_ref):
    g, n, d = x_ref.shape
    o = w_ref.shape[1]
    x = x_ref[...].reshape(g * n, d)
    xw = jnp.dot(x, w_ref[...], preferred_element_type=jnp.float32)
    ah = adj_ref[...].astype(jnp.bfloat16) + diag_ref[0][None, :]
    deg = jnp.dot(ah, k_ref[...], preferred_element_type=jnp.float32)
    s = jax.lax.rsqrt(deg).astype(jnp.bfloat16)
    vj = jnp.dot(s, mj_ref[...], preferred_element_type=jnp.float32)
    ahn = ah * vj.astype(jnp.bfloat16)
    norm3 = ahn.reshape(g, n, n)
    xw3 = xw.reshape(g, n, o).astype(jnp.bfloat16)
    agg = jax.lax.dot_general(
        norm3, xw3, (((2,), (1,)), ((0,), (0,))),
        preferred_element_type=jnp.float32)
    og = agg.astype(jnp.bfloat16).reshape(g, n * o)
    vi = jnp.dot(s, mo_ref[...], preferred_element_type=jnp.float32)
    out_ref[...] = og.astype(jnp.float32) * vi + bflat_ref[0][None, :]


def kernel(node_states, adj, W_gnn, b_gnn):
    b, n, d = node_states.shape
    o = W_gnn.shape[1]
    nn, no = n * n, n * o
    cc = jnp.arange(nn, dtype=jnp.int32)
    kk = jnp.arange(n, dtype=jnp.int32)
    co = jnp.arange(no, dtype=jnp.int32)
    diag = (cc // n == cc % n).astype(jnp.bfloat16).reshape(1, nn)
    k_mat = (cc[:, None] // n == kk[None, :]).astype(jnp.bfloat16)
    mj = (kk[:, None] == cc[None, :] % n).astype(jnp.bfloat16)
    mo = (kk[:, None] == co[None, :] // o).astype(jnp.bfloat16)
    b_flat = jnp.tile(b_gnn, n).reshape(1, no)
    out = pl.pallas_call(
        _gcn_block,
        grid=(b // _G,),
        in_specs=[
            pl.BlockSpec((_G, n, d), lambda i: (i, 0, 0)),
            pl.BlockSpec((_G, nn), lambda i: (i, 0)),
            pl.BlockSpec((d, o), lambda i: (0, 0)),
            pl.BlockSpec((1, nn), lambda i: (0, 0)),
            pl.BlockSpec((nn, n), lambda i: (0, 0)),
            pl.BlockSpec((n, nn), lambda i: (0, 0)),
            pl.BlockSpec((n, no), lambda i: (0, 0)),
            pl.BlockSpec((1, no), lambda i: (0, 0)),
        ],
        out_specs=pl.BlockSpec((_G, no), lambda i: (i, 0)),
        out_shape=jax.ShapeDtypeStruct((b, no), jnp.float32),
        compiler_params=pltpu.CompilerParams(
            dimension_semantics=("parallel",)),
    )(node_states, adj.reshape(b, nn), W_gnn, diag, k_mat, mj, mo, b_flat)
    return out
